# baseline (speedup 1.0000x reference)
import jax
import jax.numpy as jnp
from jax import lax
from jax.experimental import pallas as pl
from jax.experimental.pallas import tpu as pltpu
from jax.experimental.pallas import tpu_sc as plsc

N_VOCAB = 1_000_000
D_MODEL = 64
ROW = 128
N_BATCH = 4096
SEQ = 200
NUM_WORKERS = 32
ROWS_PER_W = N_BATCH // NUM_WORKERS
NSLOT = 4
N_GROUPS = ROWS_PER_W // NSLOT

TP_BLK = 2048
TP_GRID = -(-N_VOCAB // TP_BLK)


def _tp_body(wt_ref, out_ref):
    t = jnp.swapaxes(wt_ref[...], 0, 1)
    out_ref[...] = jnp.concatenate([t, jnp.zeros_like(t)], axis=1)


@jax.jit
def _transpose_pad(w_t):
    return pl.pallas_call(
        _tp_body,
        grid=(TP_GRID,),
        in_specs=[pl.BlockSpec((D_MODEL, TP_BLK), lambda i: (0, i))],
        out_specs=pl.BlockSpec((TP_BLK, ROW), lambda i: (i, 0)),
        out_shape=jax.ShapeDtypeStruct((N_VOCAB, ROW), jnp.float32),
    )(w_t)


def _body(table_hbm, idx_hbm, out_hbm, *refs):
    idx_v = refs[0:NSLOT]
    rows_v = refs[NSLOT:2 * NSLOT]
    isem = refs[2 * NSLOT:3 * NSLOT]
    gsem = refs[3 * NSLOT:4 * NSLOT]
    osem = refs[4 * NSLOT:5 * NSLOT]

    wid = lax.axis_index("s") * 2 + lax.axis_index("c")
    base = wid * ROWS_PER_W

    def start_idx(g, b):
        pltpu.async_copy(idx_hbm.at[pl.ds((base + g) * SEQ, SEQ)],
                         idx_v[b], isem[b])

    def start_gather(b):
        pltpu.async_copy(table_hbm.at[idx_v[b]], rows_v[b], gsem[b])

    def start_store(g, b):
        pltpu.async_copy(rows_v[b], out_hbm.at[base + g], osem[b])

    def wait_idx(b):
        pltpu.make_async_copy(idx_hbm.at[pl.ds(0, SEQ)], idx_v[b],
                              isem[b]).wait()

    def wait_rows(sem, b):
        pltpu.make_async_copy(table_hbm.at[pl.ds(0, SEQ)], rows_v[b],
                              sem[b]).wait()

    for b in range(NSLOT):
        start_idx(b, b)
    for b in range(NSLOT):
        wait_idx(b)
        start_gather(b)

    def group(p, carry):
        g0 = p * NSLOT
        for b in range(NSLOT):
            wait_rows(gsem, b)
            start_store(g0 + b, b)
            start_idx(g0 + b + NSLOT, b)
        for b in range(NSLOT):
            wait_idx(b)
            wait_rows(osem, b)
            start_gather(b)
        return carry

    lax.fori_loop(0, N_GROUPS - 1, group, 0)

    g0 = (N_GROUPS - 1) * NSLOT
    for b in range(NSLOT):
        wait_rows(gsem, b)
        start_store(g0 + b, b)
    for b in range(NSLOT):
        wait_rows(osem, b)


@jax.jit
def _embed(table, flat_idx):
    mesh = plsc.VectorSubcoreMesh(core_axis_name="c", subcore_axis_name="s")
    scratch = (
        [pltpu.VMEM((SEQ,), jnp.int32) for _ in range(NSLOT)]
        + [pltpu.VMEM((SEQ, ROW), jnp.float32) for _ in range(NSLOT)]
        + [pltpu.SemaphoreType.DMA for _ in range(3 * NSLOT)]
    )
    f = pl.kernel(
        _body,
        out_type=jax.ShapeDtypeStruct((N_BATCH, SEQ, ROW), jnp.float32),
        mesh=mesh,
        scratch_types=scratch,
        compiler_params=pltpu.CompilerParams(use_tc_tiling_on_sc=False),
    )
    return f(table, flat_idx)


def kernel(x, weight):
    flat = x.reshape(-1).astype(jnp.int32)
    table = _transpose_pad(lax.optimization_barrier(weight.T))
    out = _embed(table, flat)
    return lax.slice(out, (0, 0, 0), (N_BATCH, SEQ, D_MODEL))


# speedup vs baseline: 1.4171x; 1.4171x over previous
import jax
import jax.numpy as jnp
from jax import lax
from jax.experimental import pallas as pl
from jax.experimental.pallas import tpu as pltpu
from jax.experimental.pallas import tpu_sc as plsc

N_VOCAB = 1_000_000
D_MODEL = 64
ROW = 128
N_BATCH = 4096
SEQ = 200
NUM_WORKERS = 32
ROWS_PER_W = N_BATCH // NUM_WORKERS
NSLOT = 4
N_GROUPS = ROWS_PER_W // NSLOT

TP_BLK = 4096
TP_GRID = -(-N_VOCAB // TP_BLK)
V_PAD = TP_GRID * TP_BLK


def _tp_body(wt_ref, out_ref):
    t = jnp.swapaxes(wt_ref[...], 0, 1)
    out_ref[...] = jnp.concatenate(
        [t[: TP_BLK // 2], t[TP_BLK // 2:]], axis=1)


@jax.jit
def _transpose_pack(w_t):
    return pl.pallas_call(
        _tp_body,
        grid=(TP_GRID,),
        in_specs=[pl.BlockSpec((D_MODEL, TP_BLK), lambda i: (0, i))],
        out_specs=pl.BlockSpec((TP_BLK // 2, ROW), lambda i: (i, 0)),
        out_shape=jax.ShapeDtypeStruct((V_PAD // 2, ROW), jnp.float32),
    )(w_t)


def _body(table_hbm, idx_hbm, out_hbm, *refs):
    idx_v = refs[0:NSLOT]
    rows_v = refs[NSLOT:2 * NSLOT]
    isem = refs[2 * NSLOT:3 * NSLOT]
    gsem = refs[3 * NSLOT:4 * NSLOT]
    osem = refs[4 * NSLOT:5 * NSLOT]

    wid = lax.axis_index("s") * 2 + lax.axis_index("c")
    base = wid * ROWS_PER_W

    def start_idx(g, b):
        pltpu.async_copy(idx_hbm.at[pl.ds((base + g) * SEQ, SEQ)],
                         idx_v[b], isem[b])

    def start_gather(b):
        pltpu.async_copy(table_hbm.at[idx_v[b]], rows_v[b], gsem[b])

    def start_store(g, b):
        pltpu.async_copy(
            rows_v[b],
            out_hbm.at[base + g, pl.ds(0, SEQ), pl.ds(0, D_MODEL)],
            osem[b])

    def wait_idx(b):
        pltpu.make_async_copy(idx_hbm.at[pl.ds(0, SEQ)], idx_v[b],
                              isem[b]).wait()

    def wait_rows(sem, b):
        pltpu.make_async_copy(table_hbm.at[pl.ds(0, SEQ)], rows_v[b],
                              sem[b]).wait()

    for b in range(NSLOT):
        start_idx(b, b)
    for b in range(NSLOT):
        wait_idx(b)
        start_gather(b)

    def group(p, carry):
        g0 = p * NSLOT
        for b in range(NSLOT):
            wait_rows(gsem, b)
            start_store(g0 + b, b)
            start_idx(g0 + b + NSLOT, b)
        for b in range(NSLOT):
            wait_idx(b)
            wait_rows(osem, b)
            start_gather(b)
        return carry

    lax.fori_loop(0, N_GROUPS - 1, group, 0)

    g0 = (N_GROUPS - 1) * NSLOT
    for b in range(NSLOT):
        wait_rows(gsem, b)
        start_store(g0 + b, b)
    for b in range(NSLOT):
        wait_rows(osem, b)


@jax.jit
def _embed(table, flat_idx):
    mesh = plsc.VectorSubcoreMesh(core_axis_name="c", subcore_axis_name="s")
    scratch = (
        [pltpu.VMEM((SEQ,), jnp.int32) for _ in range(NSLOT)]
        + [pltpu.VMEM((SEQ, D_MODEL), jnp.float32) for _ in range(NSLOT)]
        + [pltpu.SemaphoreType.DMA for _ in range(3 * NSLOT)]
    )
    f = pl.kernel(
        _body,
        out_type=jax.ShapeDtypeStruct((N_BATCH, SEQ, ROW), jnp.float32),
        mesh=mesh,
        scratch_types=scratch,
        compiler_params=pltpu.CompilerParams(use_tc_tiling_on_sc=False),
    )
    return f(table, flat_idx)


def kernel(x, weight):
    flat = x.reshape(-1).astype(jnp.int32)
    flat = (flat & ~jnp.int32(TP_BLK - 1)) \
        + 2 * (flat & jnp.int32(TP_BLK // 2 - 1)) \
        + ((flat >> jnp.int32(11)) & jnp.int32(1))
    packed = _transpose_pack(lax.optimization_barrier(weight.T))
    table = packed.reshape(V_PAD, D_MODEL)
    out = _embed(table, flat)
    return lax.slice(out, (0, 0, 0), (N_BATCH, SEQ, D_MODEL))


# speedup vs baseline: 1.6706x; 1.1789x over previous
import jax
import jax.numpy as jnp
from jax import lax
from jax.experimental import pallas as pl
from jax.experimental.pallas import tpu as pltpu
from jax.experimental.pallas import tpu_sc as plsc

N_VOCAB = 1_000_000
D_MODEL = 64
ROW = 128
N_BATCH = 4096
SEQ = 200
NUM_WORKERS = 32
ROWS_PER_W = N_BATCH // NUM_WORKERS
NSLOT = 4
N_GROUPS = ROWS_PER_W // NSLOT

TP_BLK = 16384
TP_GRID = -(-N_VOCAB // TP_BLK)
V_PAD = TP_GRID * TP_BLK


def _tp_body(wt_ref, out_ref):
    t = jnp.swapaxes(wt_ref[...], 0, 1)
    out_ref[...] = jnp.concatenate(
        [t[: TP_BLK // 2], t[TP_BLK // 2:]], axis=1)


@jax.jit
def _transpose_pack(w_t):
    return pl.pallas_call(
        _tp_body,
        grid=(TP_GRID,),
        in_specs=[pl.BlockSpec((D_MODEL, TP_BLK), lambda i: (0, i))],
        out_specs=pl.BlockSpec((TP_BLK // 2, ROW), lambda i: (i, 0)),
        out_shape=jax.ShapeDtypeStruct((V_PAD // 2, ROW), jnp.float32),
    )(w_t)


def _body(table_hbm, idx_hbm, out_hbm, *refs):
    idx_v = refs[0:NSLOT]
    rows_v = refs[NSLOT:2 * NSLOT]
    isem = refs[2 * NSLOT:3 * NSLOT]
    gsem = refs[3 * NSLOT:4 * NSLOT]
    osem = refs[4 * NSLOT:5 * NSLOT]

    wid = lax.axis_index("s") * 2 + lax.axis_index("c")
    base = wid * ROWS_PER_W

    def start_idx(g, b):
        pltpu.async_copy(idx_hbm.at[pl.ds((base + g) * SEQ, SEQ)],
                         idx_v[b], isem[b])

    def start_gather(b):
        pltpu.async_copy(table_hbm.at[idx_v[b]], rows_v[b], gsem[b])

    def start_store(g, b):
        pltpu.async_copy(
            rows_v[b],
            out_hbm.at[base + g, pl.ds(0, SEQ), pl.ds(0, D_MODEL)],
            osem[b])

    def wait_idx(b):
        pltpu.make_async_copy(idx_hbm.at[pl.ds(0, SEQ)], idx_v[b],
                              isem[b]).wait()

    def wait_rows(sem, b):
        pltpu.make_async_copy(table_hbm.at[pl.ds(0, SEQ)], rows_v[b],
                              sem[b]).wait()

    for b in range(NSLOT):
        start_idx(b, b)
    for b in range(NSLOT):
        wait_idx(b)
        start_gather(b)

    def group(p, carry):
        g0 = p * NSLOT
        for b in range(NSLOT):
            wait_rows(gsem, b)
            start_store(g0 + b, b)
            start_idx(g0 + b + NSLOT, b)
        for b in range(NSLOT):
            wait_idx(b)
            wait_rows(osem, b)
            start_gather(b)
        return carry

    lax.fori_loop(0, N_GROUPS - 1, group, 0)

    g0 = (N_GROUPS - 1) * NSLOT
    for b in range(NSLOT):
        wait_rows(gsem, b)
        start_store(g0 + b, b)
    for b in range(NSLOT):
        wait_rows(osem, b)


@jax.jit
def _embed(table, flat_idx):
    mesh = plsc.VectorSubcoreMesh(core_axis_name="c", subcore_axis_name="s")
    scratch = (
        [pltpu.VMEM((SEQ,), jnp.int32) for _ in range(NSLOT)]
        + [pltpu.VMEM((SEQ, D_MODEL), jnp.float32) for _ in range(NSLOT)]
        + [pltpu.SemaphoreType.DMA for _ in range(3 * NSLOT)]
    )
    f = pl.kernel(
        _body,
        out_type=jax.ShapeDtypeStruct((N_BATCH, SEQ, ROW), jnp.float32),
        mesh=mesh,
        scratch_types=scratch,
        compiler_params=pltpu.CompilerParams(use_tc_tiling_on_sc=False),
    )
    return f(table, flat_idx)


def kernel(x, weight):
    flat = x.reshape(-1).astype(jnp.int32)
    half_shift = TP_BLK.bit_length() - 2
    flat = (flat & ~jnp.int32(TP_BLK - 1)) \
        + 2 * (flat & jnp.int32(TP_BLK // 2 - 1)) \
        + ((flat >> jnp.int32(half_shift)) & jnp.int32(1))
    packed = _transpose_pack(lax.optimization_barrier(weight.T))
    table = packed.reshape(V_PAD, D_MODEL)
    out = _embed(table, flat)
    return lax.slice(out, (0, 0, 0), (N_BATCH, SEQ, D_MODEL))


# speedup vs baseline: 1.7155x; 1.0269x over previous
import jax
import jax.numpy as jnp
from jax import lax
from jax.experimental import pallas as pl
from jax.experimental.pallas import tpu as pltpu
from jax.experimental.pallas import tpu_sc as plsc

N_VOCAB = 1_000_000
D_MODEL = 64
ROW = 128
N_BATCH = 4096
SEQ = 200
NUM_WORKERS = 32
ROWS_PER_W = N_BATCH // NUM_WORKERS
NSLOT = 4
N_GROUPS = ROWS_PER_W // NSLOT

TP_BLK = 32768
TP_GRID = -(-N_VOCAB // TP_BLK)
V_PAD = TP_GRID * TP_BLK


def _tp_body(wt_ref, out_ref):
    t = jnp.swapaxes(wt_ref[...], 0, 1)
    out_ref[...] = jnp.concatenate(
        [t[: TP_BLK // 2], t[TP_BLK // 2:]], axis=1)


@jax.jit
def _transpose_pack(w_t):
    return pl.pallas_call(
        _tp_body,
        grid=(TP_GRID,),
        in_specs=[pl.BlockSpec((D_MODEL, TP_BLK), lambda i: (0, i))],
        out_specs=pl.BlockSpec((TP_BLK // 2, ROW), lambda i: (i, 0)),
        out_shape=jax.ShapeDtypeStruct((V_PAD // 2, ROW), jnp.float32),
    )(w_t)


def _body(table_hbm, idx_hbm, out_hbm, *refs):
    idx_v = refs[0:NSLOT]
    rows_v = refs[NSLOT:2 * NSLOT]
    isem = refs[2 * NSLOT:3 * NSLOT]
    gsem = refs[3 * NSLOT:4 * NSLOT]
    osem = refs[4 * NSLOT:5 * NSLOT]

    wid = lax.axis_index("s") * 2 + lax.axis_index("c")
    base = wid * ROWS_PER_W

    def start_idx(g, b):
        pltpu.async_copy(idx_hbm.at[pl.ds((base + g) * SEQ, SEQ)],
                         idx_v[b], isem[b])

    def start_gather(b):
        pltpu.async_copy(table_hbm.at[idx_v[b]], rows_v[b], gsem[b])

    def start_store(g, b):
        pltpu.async_copy(
            rows_v[b],
            out_hbm.at[base + g, pl.ds(0, SEQ), pl.ds(0, D_MODEL)],
            osem[b])

    def wait_idx(b):
        pltpu.make_async_copy(idx_hbm.at[pl.ds(0, SEQ)], idx_v[b],
                              isem[b]).wait()

    def wait_rows(sem, b):
        pltpu.make_async_copy(table_hbm.at[pl.ds(0, SEQ)], rows_v[b],
                              sem[b]).wait()

    for b in range(NSLOT):
        start_idx(b, b)
    for b in range(NSLOT):
        wait_idx(b)
        start_gather(b)

    def group(p, carry):
        g0 = p * NSLOT
        for b in range(NSLOT):
            wait_rows(gsem, b)
            start_store(g0 + b, b)
            start_idx(g0 + b + NSLOT, b)
        for b in range(NSLOT):
            wait_idx(b)
            wait_rows(osem, b)
            start_gather(b)
        return carry

    lax.fori_loop(0, N_GROUPS - 1, group, 0)

    g0 = (N_GROUPS - 1) * NSLOT
    for b in range(NSLOT):
        wait_rows(gsem, b)
        start_store(g0 + b, b)
    for b in range(NSLOT):
        wait_rows(osem, b)


@jax.jit
def _embed(table, flat_idx):
    mesh = plsc.VectorSubcoreMesh(core_axis_name="c", subcore_axis_name="s")
    scratch = (
        [pltpu.VMEM((SEQ,), jnp.int32) for _ in range(NSLOT)]
        + [pltpu.VMEM((SEQ, D_MODEL), jnp.float32) for _ in range(NSLOT)]
        + [pltpu.SemaphoreType.DMA for _ in range(3 * NSLOT)]
    )
    f = pl.kernel(
        _body,
        out_type=jax.ShapeDtypeStruct((N_BATCH, SEQ, ROW), jnp.float32),
        mesh=mesh,
        scratch_types=scratch,
        compiler_params=pltpu.CompilerParams(use_tc_tiling_on_sc=False),
    )
    return f(table, flat_idx)


def kernel(x, weight):
    flat = x.reshape(-1).astype(jnp.int32)
    half_shift = TP_BLK.bit_length() - 2
    flat = (flat & ~jnp.int32(TP_BLK - 1)) \
        + 2 * (flat & jnp.int32(TP_BLK // 2 - 1)) \
        + ((flat >> jnp.int32(half_shift)) & jnp.int32(1))
    packed = _transpose_pack(lax.optimization_barrier(weight.T))
    table = packed.reshape(V_PAD, D_MODEL)
    out = _embed(table, flat)
    return lax.slice(out, (0, 0, 0), (N_BATCH, SEQ, D_MODEL))
